# baseline (speedup 1.0000x reference)
import functools

import jax
import jax.numpy as jnp
from jax import lax
from jax.experimental import pallas as pl
from jax.experimental.pallas import tpu as pltpu
from jax.experimental.pallas import tpu_sc as plsc

_D = 1024
_E = 8
_K = 2
_H = 2752
_T = 2048
_BLK = 128
_P = _T * _K + _E * _BLK
_NB = _P // _BLK
_NH = 2
_HH = _H // _NH


_CHUNK = 128
_NCHUNK = _T // _CHUNK


def _meta_body(x_ref, wr_ref, dsta_ref, dstb_ref, wa_ref, wb_ref, be_ref):
    logits = jax.lax.dot_general(
        x_ref[...].astype(jnp.bfloat16), wr_ref[...].astype(jnp.bfloat16),
        (((1,), (1,)), ((), ())),
        preferred_element_type=jnp.float32)

    lane = jax.lax.broadcasted_iota(jnp.int32, (_T, _E), 1)
    v1 = jnp.max(logits, axis=1, keepdims=True)
    i1 = jnp.min(jnp.where(logits == v1, lane, _E), axis=1,
                 keepdims=True)
    masked = jnp.where(lane == i1, -jnp.inf, logits)
    v2 = jnp.max(masked, axis=1, keepdims=True)
    i2 = jnp.min(jnp.where(masked == v2, lane, _E), axis=1,
                 keepdims=True)
    wa = 1.0 / (1.0 + jnp.exp(v2 - v1))
    wa_ref[...] = jnp.broadcast_to(wa, (_T, 16))
    wb_ref[...] = jnp.broadcast_to(1.0 - wa, (_T, 16))

    oha = (lane == i1)
    ohb = (lane == i2)
    s = (oha.astype(jnp.float32) + ohb.astype(jnp.float32))
    r = jax.lax.broadcasted_iota(jnp.int32, (_CHUNK, _CHUNK), 0)
    c = jax.lax.broadcasted_iota(jnp.int32, (_CHUNK, _CHUNK), 1)
    tri = (c < r).astype(jnp.bfloat16)
    parts = []
    run = jnp.zeros((1, _E), jnp.float32)
    for ci in range(_NCHUNK):
        sc_ = jax.lax.slice_in_dim(s, ci * _CHUNK, (ci + 1) * _CHUNK, axis=0)
        intra = jax.lax.dot_general(
            tri, sc_.astype(jnp.bfloat16), (((1,), (0,)), ((), ())),
            preferred_element_type=jnp.float32)
        parts.append(intra + run)
        run = run + jnp.sum(sc_, axis=0, keepdims=True)
    cexcl = jnp.concatenate(parts, axis=0)
    gsz = run

    padded = jnp.ceil(gsz / _BLK) * _BLK
    el = jax.lax.broadcasted_iota(jnp.int32, (_E, _E), 1)
    er = jax.lax.broadcasted_iota(jnp.int32, (_E, _E), 0)
    pstart = jnp.sum(jnp.where(el < er, jnp.broadcast_to(padded, (_E, _E)),
                               0.0), axis=1, keepdims=True)
    pstart_row = jnp.broadcast_to(pstart.reshape(1, _E), (_T, _E))

    ranka = jnp.sum(jnp.where(oha, cexcl, 0.0), axis=1, keepdims=True)
    rankb = jnp.sum(jnp.where(ohb, cexcl + oha.astype(jnp.float32), 0.0),
                    axis=1, keepdims=True)
    psa = jnp.sum(jnp.where(oha, pstart_row, 0.0), axis=1, keepdims=True)
    psb = jnp.sum(jnp.where(ohb, pstart_row, 0.0), axis=1, keepdims=True)
    dsta = (psa + ranka).astype(jnp.int32)
    dstb = (psb + rankb).astype(jnp.int32)
    dsta_ref[...] = dsta
    dstb_ref[...] = dstb

    blk = jax.lax.broadcasted_iota(jnp.int32, (_NB, _E), 0) * _BLK
    psrow = jnp.broadcast_to(pstart.reshape(1, _E), (_NB, _E))
    be_ref[...] = (jnp.sum((psrow <= blk.astype(jnp.float32))
                           .astype(jnp.int32), axis=1, keepdims=True) - 1)


def _router_meta(x_flat, Wr):
    return pl.pallas_call(
        _meta_body,
        out_shape=(
            jax.ShapeDtypeStruct((_T, 1), jnp.int32),
            jax.ShapeDtypeStruct((_T, 1), jnp.int32),
            jax.ShapeDtypeStruct((_T, 16), jnp.float32),
            jax.ShapeDtypeStruct((_T, 16), jnp.float32),
            jax.ShapeDtypeStruct((_NB, 1), jnp.int32),
        ),
    )(x_flat, Wr)


_NT = (((1,), (1,)), ((), ()))


def _f1_body(be_ref, xs_ref, w1_ref, w3_ref, g_ref):
    x = xs_ref[...].astype(jnp.bfloat16)
    w1 = w1_ref[0].astype(jnp.bfloat16)
    w3 = w3_ref[0].astype(jnp.bfloat16)
    h1 = jax.lax.dot_general(x, w1, _NT, preferred_element_type=jnp.float32)
    h3 = jax.lax.dot_general(x, w3, _NT, preferred_element_type=jnp.float32)
    g_ref[...] = (h1 * (1.0 / (1.0 + jnp.exp(-h1))) * h3).astype(jnp.bfloat16)


def _f2_body(be_ref, g_ref, w2_ref, out_ref):
    g = g_ref[...]
    w2 = w2_ref[0].astype(jnp.bfloat16)
    out_ref[...] = jax.lax.dot_general(
        g, w2, (((1,), (0,)), ((), ())), preferred_element_type=jnp.float32)


def _ffn(block_expert, xs, W1, W2, W3):
    gs1 = pltpu.PrefetchScalarGridSpec(
        num_scalar_prefetch=1,
        grid=(_NB,),
        in_specs=[
            pl.BlockSpec((_BLK, _D), lambda i, be: (i, 0)),
            pl.BlockSpec((1, _H, _D), lambda i, be: (be[i], 0, 0)),
            pl.BlockSpec((1, _H, _D), lambda i, be: (be[i], 0, 0)),
        ],
        out_specs=pl.BlockSpec((_BLK, _H), lambda i, be: (i, 0)),
    )
    g = pl.pallas_call(
        _f1_body,
        grid_spec=gs1,
        out_shape=jax.ShapeDtypeStruct((_P, _H), jnp.bfloat16),
        compiler_params=pltpu.CompilerParams(
            dimension_semantics=("arbitrary",)),
    )(block_expert, xs, W1, W3)

    gs2 = pltpu.PrefetchScalarGridSpec(
        num_scalar_prefetch=1,
        grid=(_NB,),
        in_specs=[
            pl.BlockSpec((_BLK, _H), lambda i, be: (i, 0)),
            pl.BlockSpec((1, _H, _D), lambda i, be: (be[i], 0, 0)),
        ],
        out_specs=pl.BlockSpec((_BLK, _D), lambda i, be: (i, 0)),
    )
    return pl.pallas_call(
        _f2_body,
        grid_spec=gs2,
        out_shape=jax.ShapeDtypeStruct((_P, _D), jnp.float32),
        compiler_params=pltpu.CompilerParams(
            dimension_semantics=("arbitrary",)),
    )(block_expert, g, jnp.swapaxes(W2, 1, 2))


_NC = 2
_NS = 16
_NW = _NC * _NS
_TPW = _T // _NW
_CH = _TPW // 2

_sc_mesh = plsc.VectorSubcoreMesh(core_axis_name="c", subcore_axis_name="s")


def _sc_scatter_body(x_hbm, dstA_hbm, dstB_hbm, xs_hbm,
                     idxA_v, idxB_v, rows_v, sem):
    wid = lax.axis_index("s") * _NC + lax.axis_index("c")
    base = wid * _TPW
    pltpu.sync_copy(dstA_hbm.at[pl.ds(base, _TPW)], idxA_v)
    pltpu.sync_copy(dstB_hbm.at[pl.ds(base, _TPW)], idxB_v)
    pltpu.sync_copy(x_hbm.at[pl.ds(base, _TPW)], rows_v)
    cpA = pltpu.make_async_copy(rows_v, xs_hbm.at[idxA_v], sem)
    cpB = pltpu.make_async_copy(rows_v, xs_hbm.at[idxB_v], sem)
    cpA.start()
    cpB.start()
    cpA.wait()
    cpB.wait()


@functools.partial(
    pl.kernel,
    out_type=jax.ShapeDtypeStruct((_P, _D), jnp.float32),
    mesh=_sc_mesh,
    scratch_types=[
        pltpu.VMEM((_TPW,), jnp.int32),
        pltpu.VMEM((_TPW,), jnp.int32),
        pltpu.VMEM((_TPW, _D), jnp.float32),
        pltpu.SemaphoreType.DMA,
    ],
)
def _sc_scatter(x_hbm, dstA_hbm, dstB_hbm, xs_hbm, idxA_v, idxB_v, rows_v, sem):
    _sc_scatter_body(x_hbm, dstA_hbm, dstB_hbm, xs_hbm,
                     idxA_v, idxB_v, rows_v, sem)


def _sc_combine_body(ys_hbm, dstA_hbm, dstB_hbm, wA_hbm, wB_hbm, out_hbm,
                     idxA_v, idxB_v, wA_v, wB_v, bufA_v, bufB_v, out_v, sem):
    wid = lax.axis_index("s") * _NC + lax.axis_index("c")
    base = wid * _TPW
    for half in range(_TPW // _CH):
        hb = base + half * _CH
        pltpu.sync_copy(dstA_hbm.at[pl.ds(hb, _CH)], idxA_v)
        pltpu.sync_copy(dstB_hbm.at[pl.ds(hb, _CH)], idxB_v)
        pltpu.sync_copy(wA_hbm.at[pl.ds(hb, _CH)], wA_v)
        pltpu.sync_copy(wB_hbm.at[pl.ds(hb, _CH)], wB_v)
        cpA = pltpu.make_async_copy(ys_hbm.at[idxA_v], bufA_v, sem)
        cpB = pltpu.make_async_copy(ys_hbm.at[idxB_v], bufB_v, sem)
        cpA.start()
        cpB.start()
        cpA.wait()
        cpB.wait()

        def tok(i, _):
            wa = wA_v[i, :]
            wb = wB_v[i, :]
            for k in range(_D // 16):
                a = bufA_v[i, pl.ds(k * 16, 16)]
                b = bufB_v[i, pl.ds(k * 16, 16)]
                out_v[i, pl.ds(k * 16, 16)] = wa * a + wb * b
            return 0

        lax.fori_loop(0, _CH, tok, 0)
        pltpu.sync_copy(out_v, out_hbm.at[pl.ds(hb, _CH)])


@functools.partial(
    pl.kernel,
    out_type=jax.ShapeDtypeStruct((_T, _D), jnp.float32),
    mesh=_sc_mesh,
    scratch_types=[
        pltpu.VMEM((_CH,), jnp.int32),
        pltpu.VMEM((_CH,), jnp.int32),
        pltpu.VMEM((_CH, 16), jnp.float32),
        pltpu.VMEM((_CH, 16), jnp.float32),
        pltpu.VMEM((_CH, _D), jnp.float32),
        pltpu.VMEM((_CH, _D), jnp.float32),
        pltpu.VMEM((_CH, _D), jnp.float32),
        pltpu.SemaphoreType.DMA,
    ],
)
def _sc_combine(ys_hbm, dstA_hbm, dstB_hbm, wA_hbm, wB_hbm, out_hbm,
                idxA_v, idxB_v, wA_v, wB_v, bufA_v, bufB_v, out_v, sem):
    _sc_combine_body(ys_hbm, dstA_hbm, dstB_hbm, wA_hbm, wB_hbm, out_hbm,
                     idxA_v, idxB_v, wA_v, wB_v, bufA_v, bufB_v, out_v, sem)


def kernel(x, Wr, W1, W2, W3):
    x_flat = x.reshape(_T, _D)
    dstA2, dstB2, wA, wB, be2 = _router_meta(x_flat, Wr)
    dstA = dstA2.reshape(_T)
    dstB = dstB2.reshape(_T)
    block_expert = be2.reshape(_NB)

    xs = _sc_scatter(x_flat, dstA, dstB)
    ys = _ffn(block_expert, xs, W1, W2, W3)
    out = _sc_combine(ys, dstA, dstB, wA, wB)
    return out.reshape(1, _T, _D)


# speedup vs baseline: 1.2979x; 1.2979x over previous
import functools

import jax
import jax.numpy as jnp
from jax import lax
from jax.experimental import pallas as pl
from jax.experimental.pallas import tpu as pltpu
from jax.experimental.pallas import tpu_sc as plsc

_D = 1024
_E = 8
_K = 2
_H = 2752
_T = 2048
_BLK = 256
_P = _T * _K + _E * _BLK
_NB = _P // _BLK
_NH = 2
_HH = _H // _NH


_CHUNK = 256
_NCHUNK = _T // _CHUNK


def _meta_body(x_ref, wr_ref, dsta_ref, dstb_ref, wa_ref, wb_ref, be_ref):
    logits = jax.lax.dot_general(
        x_ref[...].astype(jnp.bfloat16), wr_ref[...].astype(jnp.bfloat16),
        (((1,), (1,)), ((), ())),
        preferred_element_type=jnp.float32)

    lane = jax.lax.broadcasted_iota(jnp.int32, (_T, _E), 1)
    v1 = jnp.max(logits, axis=1, keepdims=True)
    i1 = jnp.min(jnp.where(logits == v1, lane, _E), axis=1,
                 keepdims=True)
    masked = jnp.where(lane == i1, -jnp.inf, logits)
    v2 = jnp.max(masked, axis=1, keepdims=True)
    i2 = jnp.min(jnp.where(masked == v2, lane, _E), axis=1,
                 keepdims=True)
    wa = 1.0 / (1.0 + jnp.exp(v2 - v1))
    wa_ref[...] = jnp.broadcast_to(wa, (_T, 16))
    wb_ref[...] = jnp.broadcast_to(1.0 - wa, (_T, 16))

    oha = (lane == i1)
    ohb = (lane == i2)
    s = (oha.astype(jnp.float32) + ohb.astype(jnp.float32))
    r = jax.lax.broadcasted_iota(jnp.int32, (_CHUNK, _CHUNK), 0)
    c = jax.lax.broadcasted_iota(jnp.int32, (_CHUNK, _CHUNK), 1)
    tri = (c < r).astype(jnp.bfloat16)
    parts = []
    run = jnp.zeros((1, _E), jnp.float32)
    for ci in range(_NCHUNK):
        sc_ = jax.lax.slice_in_dim(s, ci * _CHUNK, (ci + 1) * _CHUNK, axis=0)
        intra = jax.lax.dot_general(
            tri, sc_.astype(jnp.bfloat16), (((1,), (0,)), ((), ())),
            preferred_element_type=jnp.float32)
        parts.append(intra + run)
        run = run + jnp.sum(sc_, axis=0, keepdims=True)
    cexcl = jnp.concatenate(parts, axis=0)
    gsz = run

    padded = jnp.ceil(gsz / _BLK) * _BLK
    el = jax.lax.broadcasted_iota(jnp.int32, (_E, _E), 1)
    er = jax.lax.broadcasted_iota(jnp.int32, (_E, _E), 0)
    pstart = jnp.sum(jnp.where(el < er, jnp.broadcast_to(padded, (_E, _E)),
                               0.0), axis=1, keepdims=True)
    pstart_row = jnp.broadcast_to(pstart.reshape(1, _E), (_T, _E))

    ranka = jnp.sum(jnp.where(oha, cexcl, 0.0), axis=1, keepdims=True)
    rankb = jnp.sum(jnp.where(ohb, cexcl + oha.astype(jnp.float32), 0.0),
                    axis=1, keepdims=True)
    psa = jnp.sum(jnp.where(oha, pstart_row, 0.0), axis=1, keepdims=True)
    psb = jnp.sum(jnp.where(ohb, pstart_row, 0.0), axis=1, keepdims=True)
    dsta = (psa + ranka).astype(jnp.int32)
    dstb = (psb + rankb).astype(jnp.int32)
    dsta_ref[...] = dsta
    dstb_ref[...] = dstb

    blk = jax.lax.broadcasted_iota(jnp.int32, (_NB, _E), 0) * _BLK
    psrow = jnp.broadcast_to(pstart.reshape(1, _E), (_NB, _E))
    be_ref[...] = (jnp.sum((psrow <= blk.astype(jnp.float32))
                           .astype(jnp.int32), axis=1, keepdims=True) - 1)


def _router_meta(x_flat, Wr):
    return pl.pallas_call(
        _meta_body,
        out_shape=(
            jax.ShapeDtypeStruct((_T, 1), jnp.int32),
            jax.ShapeDtypeStruct((_T, 1), jnp.int32),
            jax.ShapeDtypeStruct((_T, 16), jnp.float32),
            jax.ShapeDtypeStruct((_T, 16), jnp.float32),
            jax.ShapeDtypeStruct((_NB, 1), jnp.int32),
        ),
    )(x_flat, Wr)


_NT = (((1,), (1,)), ((), ()))


def _f1_body(be_ref, xs_ref, w1_ref, w3_ref, g_ref):
    x = xs_ref[...].astype(jnp.bfloat16)
    w1 = w1_ref[0].astype(jnp.bfloat16)
    w3 = w3_ref[0].astype(jnp.bfloat16)
    h1 = jax.lax.dot_general(x, w1, _NT, preferred_element_type=jnp.float32)
    h3 = jax.lax.dot_general(x, w3, _NT, preferred_element_type=jnp.float32)
    g_ref[...] = (h1 * (1.0 / (1.0 + jnp.exp(-h1))) * h3).astype(jnp.bfloat16)


def _f2_body(be_ref, g_ref, w2_ref, out_ref):
    g = g_ref[...]
    w2 = w2_ref[0].astype(jnp.bfloat16)
    out_ref[...] = jax.lax.dot_general(
        g, w2, (((1,), (0,)), ((), ())), preferred_element_type=jnp.float32)


def _ffn(block_expert, xs, W1, W2, W3):
    gs1 = pltpu.PrefetchScalarGridSpec(
        num_scalar_prefetch=1,
        grid=(_NB,),
        in_specs=[
            pl.BlockSpec((_BLK, _D), lambda i, be: (i, 0)),
            pl.BlockSpec((1, _H, _D), lambda i, be: (be[i], 0, 0)),
            pl.BlockSpec((1, _H, _D), lambda i, be: (be[i], 0, 0)),
        ],
        out_specs=pl.BlockSpec((_BLK, _H), lambda i, be: (i, 0)),
    )
    g = pl.pallas_call(
        _f1_body,
        grid_spec=gs1,
        out_shape=jax.ShapeDtypeStruct((_P, _H), jnp.bfloat16),
        compiler_params=pltpu.CompilerParams(
            dimension_semantics=("arbitrary",)),
    )(block_expert, xs, W1, W3)

    gs2 = pltpu.PrefetchScalarGridSpec(
        num_scalar_prefetch=1,
        grid=(_NB,),
        in_specs=[
            pl.BlockSpec((_BLK, _H), lambda i, be: (i, 0)),
            pl.BlockSpec((1, _H, _D), lambda i, be: (be[i], 0, 0)),
        ],
        out_specs=pl.BlockSpec((_BLK, _D), lambda i, be: (i, 0)),
    )
    return pl.pallas_call(
        _f2_body,
        grid_spec=gs2,
        out_shape=jax.ShapeDtypeStruct((_P, _D), jnp.float32),
        compiler_params=pltpu.CompilerParams(
            dimension_semantics=("arbitrary",)),
    )(block_expert, g, jnp.swapaxes(W2, 1, 2))


_NC = 2
_NS = 16
_NW = _NC * _NS
_TPW = _T // _NW
_CH = _TPW // 2

_sc_mesh = plsc.VectorSubcoreMesh(core_axis_name="c", subcore_axis_name="s")


def _sc_scatter_body(x_hbm, dstA_hbm, dstB_hbm, xs_hbm,
                     idxA_v, idxB_v, rows_v, sem):
    wid = lax.axis_index("s") * _NC + lax.axis_index("c")
    base = wid * _TPW
    pltpu.sync_copy(dstA_hbm.at[pl.ds(base, _TPW)], idxA_v)
    pltpu.sync_copy(dstB_hbm.at[pl.ds(base, _TPW)], idxB_v)
    pltpu.sync_copy(x_hbm.at[pl.ds(base, _TPW)], rows_v)
    cpA = pltpu.make_async_copy(rows_v, xs_hbm.at[idxA_v], sem)
    cpB = pltpu.make_async_copy(rows_v, xs_hbm.at[idxB_v], sem)
    cpA.start()
    cpB.start()
    cpA.wait()
    cpB.wait()


@functools.partial(
    pl.kernel,
    out_type=jax.ShapeDtypeStruct((_P, _D), jnp.float32),
    mesh=_sc_mesh,
    scratch_types=[
        pltpu.VMEM((_TPW,), jnp.int32),
        pltpu.VMEM((_TPW,), jnp.int32),
        pltpu.VMEM((_TPW, _D), jnp.float32),
        pltpu.SemaphoreType.DMA,
    ],
)
def _sc_scatter(x_hbm, dstA_hbm, dstB_hbm, xs_hbm, idxA_v, idxB_v, rows_v, sem):
    _sc_scatter_body(x_hbm, dstA_hbm, dstB_hbm, xs_hbm,
                     idxA_v, idxB_v, rows_v, sem)


def _sc_combine_body(ys_hbm, dstA_hbm, dstB_hbm, wA_hbm, wB_hbm, out_hbm,
                     idxA_v, idxB_v, wA_v, wB_v, bufA_v, bufB_v, out_v, sem):
    wid = lax.axis_index("s") * _NC + lax.axis_index("c")
    base = wid * _TPW
    for half in range(_TPW // _CH):
        hb = base + half * _CH
        pltpu.sync_copy(dstA_hbm.at[pl.ds(hb, _CH)], idxA_v)
        pltpu.sync_copy(dstB_hbm.at[pl.ds(hb, _CH)], idxB_v)
        pltpu.sync_copy(wA_hbm.at[pl.ds(hb, _CH)], wA_v)
        pltpu.sync_copy(wB_hbm.at[pl.ds(hb, _CH)], wB_v)
        cpA = pltpu.make_async_copy(ys_hbm.at[idxA_v], bufA_v, sem)
        cpB = pltpu.make_async_copy(ys_hbm.at[idxB_v], bufB_v, sem)
        cpA.start()
        cpB.start()
        cpA.wait()
        cpB.wait()

        def tok(i, _):
            wa = wA_v[i, :]
            wb = wB_v[i, :]
            for k in range(_D // 16):
                a = bufA_v[i, pl.ds(k * 16, 16)]
                b = bufB_v[i, pl.ds(k * 16, 16)]
                out_v[i, pl.ds(k * 16, 16)] = wa * a + wb * b
            return 0

        lax.fori_loop(0, _CH, tok, 0)
        pltpu.sync_copy(out_v, out_hbm.at[pl.ds(hb, _CH)])


@functools.partial(
    pl.kernel,
    out_type=jax.ShapeDtypeStruct((_T, _D), jnp.float32),
    mesh=_sc_mesh,
    scratch_types=[
        pltpu.VMEM((_CH,), jnp.int32),
        pltpu.VMEM((_CH,), jnp.int32),
        pltpu.VMEM((_CH, 16), jnp.float32),
        pltpu.VMEM((_CH, 16), jnp.float32),
        pltpu.VMEM((_CH, _D), jnp.float32),
        pltpu.VMEM((_CH, _D), jnp.float32),
        pltpu.VMEM((_CH, _D), jnp.float32),
        pltpu.SemaphoreType.DMA,
    ],
)
def _sc_combine(ys_hbm, dstA_hbm, dstB_hbm, wA_hbm, wB_hbm, out_hbm,
                idxA_v, idxB_v, wA_v, wB_v, bufA_v, bufB_v, out_v, sem):
    _sc_combine_body(ys_hbm, dstA_hbm, dstB_hbm, wA_hbm, wB_hbm, out_hbm,
                     idxA_v, idxB_v, wA_v, wB_v, bufA_v, bufB_v, out_v, sem)


def kernel(x, Wr, W1, W2, W3):
    x_flat = x.reshape(_T, _D)
    dstA2, dstB2, wA, wB, be2 = _router_meta(x_flat, Wr)
    dstA = dstA2.reshape(_T)
    dstB = dstB2.reshape(_T)
    block_expert = be2.reshape(_NB)

    xs = _sc_scatter(x_flat, dstA, dstB)
    ys = _ffn(block_expert, xs, W1, W2, W3)
    out = _sc_combine(ys, dstA, dstB, wA, wB)
    return out.reshape(1, _T, _D)


# speedup vs baseline: 1.3113x; 1.0103x over previous
import functools

import jax
import jax.numpy as jnp
from jax import lax
from jax.experimental import pallas as pl
from jax.experimental.pallas import tpu as pltpu
from jax.experimental.pallas import tpu_sc as plsc

_D = 1024
_E = 8
_K = 2
_H = 2752
_T = 2048
_BLK = 256
_P = _T * _K + _E * _BLK
_NB = _P // _BLK
_NH = 2
_HH = _H // _NH


_CHUNK = 256
_NCHUNK = _T // _CHUNK


def _meta_body(x_ref, wr_ref, dsta_ref, dstb_ref, wa_ref, wb_ref, be_ref):
    logits = jax.lax.dot_general(
        x_ref[...].astype(jnp.bfloat16), wr_ref[...].astype(jnp.bfloat16),
        (((1,), (1,)), ((), ())),
        preferred_element_type=jnp.float32)

    lane = jax.lax.broadcasted_iota(jnp.int32, (_T, _E), 1)
    v1 = jnp.max(logits, axis=1, keepdims=True)
    i1 = jnp.min(jnp.where(logits == v1, lane, _E), axis=1,
                 keepdims=True)
    masked = jnp.where(lane == i1, -jnp.inf, logits)
    v2 = jnp.max(masked, axis=1, keepdims=True)
    i2 = jnp.min(jnp.where(masked == v2, lane, _E), axis=1,
                 keepdims=True)
    wa = 1.0 / (1.0 + jnp.exp(v2 - v1))
    wa_ref[...] = jnp.broadcast_to(wa, (_T, 16))
    wb_ref[...] = jnp.broadcast_to(1.0 - wa, (_T, 16))

    oha = (lane == i1)
    ohb = (lane == i2)
    s = (oha.astype(jnp.float32) + ohb.astype(jnp.float32))
    r = jax.lax.broadcasted_iota(jnp.int32, (_CHUNK, _CHUNK), 0)
    c = jax.lax.broadcasted_iota(jnp.int32, (_CHUNK, _CHUNK), 1)
    tri = (c < r).astype(jnp.bfloat16)
    parts = []
    run = jnp.zeros((1, _E), jnp.float32)
    for ci in range(_NCHUNK):
        sc_ = jax.lax.slice_in_dim(s, ci * _CHUNK, (ci + 1) * _CHUNK, axis=0)
        intra = jax.lax.dot_general(
            tri, sc_.astype(jnp.bfloat16), (((1,), (0,)), ((), ())),
            preferred_element_type=jnp.float32)
        parts.append(intra + run)
        run = run + jnp.sum(sc_, axis=0, keepdims=True)
    cexcl = jnp.concatenate(parts, axis=0)
    gsz = run

    padded = jnp.ceil(gsz / _BLK) * _BLK
    el = jax.lax.broadcasted_iota(jnp.int32, (_E, _E), 1)
    er = jax.lax.broadcasted_iota(jnp.int32, (_E, _E), 0)
    pstart = jnp.sum(jnp.where(el < er, jnp.broadcast_to(padded, (_E, _E)),
                               0.0), axis=1, keepdims=True)
    pstart_row = jnp.broadcast_to(pstart.reshape(1, _E), (_T, _E))

    ranka = jnp.sum(jnp.where(oha, cexcl, 0.0), axis=1, keepdims=True)
    rankb = jnp.sum(jnp.where(ohb, cexcl + oha.astype(jnp.float32), 0.0),
                    axis=1, keepdims=True)
    psa = jnp.sum(jnp.where(oha, pstart_row, 0.0), axis=1, keepdims=True)
    psb = jnp.sum(jnp.where(ohb, pstart_row, 0.0), axis=1, keepdims=True)
    dsta = (psa + ranka).astype(jnp.int32)
    dstb = (psb + rankb).astype(jnp.int32)
    dsta_ref[...] = dsta
    dstb_ref[...] = dstb

    blk = jax.lax.broadcasted_iota(jnp.int32, (_NB, _E), 0) * _BLK
    psrow = jnp.broadcast_to(pstart.reshape(1, _E), (_NB, _E))
    be_ref[...] = (jnp.sum((psrow <= blk.astype(jnp.float32))
                           .astype(jnp.int32), axis=1, keepdims=True) - 1)


def _router_meta(x_flat, Wr):
    return pl.pallas_call(
        _meta_body,
        out_shape=(
            jax.ShapeDtypeStruct((_T, 1), jnp.int32),
            jax.ShapeDtypeStruct((_T, 1), jnp.int32),
            jax.ShapeDtypeStruct((_T, 16), jnp.float32),
            jax.ShapeDtypeStruct((_T, 16), jnp.float32),
            jax.ShapeDtypeStruct((_NB, 1), jnp.int32),
        ),
    )(x_flat, Wr)


_NT = (((1,), (1,)), ((), ()))


def _f1_body(be_ref, xs_ref, w1_ref, w3_ref, g_ref):
    x = xs_ref[...].astype(jnp.bfloat16)
    w1 = w1_ref[0].astype(jnp.bfloat16)
    w3 = w3_ref[0].astype(jnp.bfloat16)
    h1 = jax.lax.dot_general(x, w1, _NT, preferred_element_type=jnp.float32)
    h3 = jax.lax.dot_general(x, w3, _NT, preferred_element_type=jnp.float32)
    g_ref[...] = (h1 * (1.0 / (1.0 + jnp.exp(-h1))) * h3).astype(jnp.bfloat16)


def _f2_body(be_ref, g_ref, w2_ref, out_ref):
    g = g_ref[...]
    w2 = w2_ref[0].astype(jnp.bfloat16)
    out_ref[...] = jax.lax.dot_general(
        g, w2, (((1,), (0,)), ((), ())), preferred_element_type=jnp.float32)


def _ffn(block_expert, xs, W1, W2, W3):
    gs1 = pltpu.PrefetchScalarGridSpec(
        num_scalar_prefetch=1,
        grid=(_NB,),
        in_specs=[
            pl.BlockSpec((_BLK, _D), lambda i, be: (i, 0)),
            pl.BlockSpec((1, _H, _D), lambda i, be: (be[i], 0, 0)),
            pl.BlockSpec((1, _H, _D), lambda i, be: (be[i], 0, 0)),
        ],
        out_specs=pl.BlockSpec((_BLK, _H), lambda i, be: (i, 0)),
    )
    g = pl.pallas_call(
        _f1_body,
        grid_spec=gs1,
        out_shape=jax.ShapeDtypeStruct((_P, _H), jnp.bfloat16),
        compiler_params=pltpu.CompilerParams(
            dimension_semantics=("arbitrary",)),
    )(block_expert, xs, W1, W3)

    gs2 = pltpu.PrefetchScalarGridSpec(
        num_scalar_prefetch=1,
        grid=(_NB,),
        in_specs=[
            pl.BlockSpec((_BLK, _H), lambda i, be: (i, 0)),
            pl.BlockSpec((1, _H, _D), lambda i, be: (be[i], 0, 0)),
        ],
        out_specs=pl.BlockSpec((_BLK, _D), lambda i, be: (i, 0)),
    )
    return pl.pallas_call(
        _f2_body,
        grid_spec=gs2,
        out_shape=jax.ShapeDtypeStruct((_P, _D), jnp.float32),
        compiler_params=pltpu.CompilerParams(
            dimension_semantics=("arbitrary",)),
    )(block_expert, g, jnp.swapaxes(W2, 1, 2))


_NC = 2
_NS = 16
_NW = _NC * _NS
_TPW = _T // _NW
_CH = _TPW // 2

_sc_mesh = plsc.VectorSubcoreMesh(core_axis_name="c", subcore_axis_name="s")


def _sc_scatter_body(x_hbm, dstA_hbm, dstB_hbm, xs_hbm,
                     idxA_v, idxB_v, rows_v, sem):
    wid = lax.axis_index("s") * _NC + lax.axis_index("c")
    base = wid * _TPW
    pltpu.sync_copy(dstA_hbm.at[pl.ds(base, _TPW)], idxA_v)
    pltpu.sync_copy(dstB_hbm.at[pl.ds(base, _TPW)], idxB_v)
    pltpu.sync_copy(x_hbm.at[pl.ds(base, _TPW)], rows_v)
    cpA = pltpu.make_async_copy(rows_v, xs_hbm.at[idxA_v], sem)
    cpB = pltpu.make_async_copy(rows_v, xs_hbm.at[idxB_v], sem)
    cpA.start()
    cpB.start()
    cpA.wait()
    cpB.wait()


@functools.partial(
    pl.kernel,
    out_type=jax.ShapeDtypeStruct((_P, _D), jnp.float32),
    mesh=_sc_mesh,
    scratch_types=[
        pltpu.VMEM((_TPW,), jnp.int32),
        pltpu.VMEM((_TPW,), jnp.int32),
        pltpu.VMEM((_TPW, _D), jnp.float32),
        pltpu.SemaphoreType.DMA,
    ],
)
def _sc_scatter(x_hbm, dstA_hbm, dstB_hbm, xs_hbm, idxA_v, idxB_v, rows_v, sem):
    _sc_scatter_body(x_hbm, dstA_hbm, dstB_hbm, xs_hbm,
                     idxA_v, idxB_v, rows_v, sem)


_CC = 16
_NCC = _TPW // _CC


def _sc_combine_body(ys_hbm, dstA_hbm, dstB_hbm, wA_hbm, wB_hbm, out_hbm,
                     idxA_v, idxB_v, wA_v, wB_v, bufA_v, bufB_v, out_v, sem):
    wid = lax.axis_index("s") * _NC + lax.axis_index("c")
    base = wid * _TPW
    pltpu.sync_copy(dstA_hbm.at[pl.ds(base, _TPW)], idxA_v)
    pltpu.sync_copy(dstB_hbm.at[pl.ds(base, _TPW)], idxB_v)
    pltpu.sync_copy(wA_hbm.at[pl.ds(base, _TPW)], wA_v)
    pltpu.sync_copy(wB_hbm.at[pl.ds(base, _TPW)], wB_v)

    def gathers(ch):
        par = ch % 2
        cpA = pltpu.make_async_copy(
            ys_hbm.at[idxA_v.at[pl.ds(ch * _CC, _CC)]], bufA_v.at[par], sem)
        cpB = pltpu.make_async_copy(
            ys_hbm.at[idxB_v.at[pl.ds(ch * _CC, _CC)]], bufB_v.at[par], sem)
        cpA.start()
        cpB.start()
        return cpA, cpB

    pend = gathers(0)
    for ch in range(_NCC):
        par = ch % 2
        pend[0].wait()
        pend[1].wait()
        if ch + 1 < _NCC:
            pend = gathers(ch + 1)

        def tok(i, _):
            wa = wA_v[ch * _CC + i, :]
            wb = wB_v[ch * _CC + i, :]
            for k in range(_D // 16):
                a = bufA_v[par, i, pl.ds(k * 16, 16)]
                b = bufB_v[par, i, pl.ds(k * 16, 16)]
                out_v[i, pl.ds(k * 16, 16)] = wa * a + wb * b
            return 0

        lax.fori_loop(0, _CC, tok, 0)
        pltpu.sync_copy(out_v, out_hbm.at[pl.ds(base + ch * _CC, _CC)])


@functools.partial(
    pl.kernel,
    out_type=jax.ShapeDtypeStruct((_T, _D), jnp.float32),
    mesh=_sc_mesh,
    scratch_types=[
        pltpu.VMEM((_TPW,), jnp.int32),
        pltpu.VMEM((_TPW,), jnp.int32),
        pltpu.VMEM((_TPW, 16), jnp.float32),
        pltpu.VMEM((_TPW, 16), jnp.float32),
        pltpu.VMEM((2, _CC, _D), jnp.float32),
        pltpu.VMEM((2, _CC, _D), jnp.float32),
        pltpu.VMEM((_CC, _D), jnp.float32),
        pltpu.SemaphoreType.DMA,
    ],
)
def _sc_combine(ys_hbm, dstA_hbm, dstB_hbm, wA_hbm, wB_hbm, out_hbm,
                idxA_v, idxB_v, wA_v, wB_v, bufA_v, bufB_v, out_v, sem):
    _sc_combine_body(ys_hbm, dstA_hbm, dstB_hbm, wA_hbm, wB_hbm, out_hbm,
                     idxA_v, idxB_v, wA_v, wB_v, bufA_v, bufB_v, out_v, sem)


def kernel(x, Wr, W1, W2, W3):
    x_flat = x.reshape(_T, _D)
    dstA2, dstB2, wA, wB, be2 = _router_meta(x_flat, Wr)
    dstA = dstA2.reshape(_T)
    dstB = dstB2.reshape(_T)
    block_expert = be2.reshape(_NB)

    xs = _sc_scatter(x_flat, dstA, dstB)
    ys = _ffn(block_expert, xs, W1, W2, W3)
    out = _sc_combine(ys, dstA, dstB, wA, wB)
    return out.reshape(1, _T, _D)


# speedup vs baseline: 1.3138x; 1.0019x over previous
import functools

import jax
import jax.numpy as jnp
from jax import lax
from jax.experimental import pallas as pl
from jax.experimental.pallas import tpu as pltpu
from jax.experimental.pallas import tpu_sc as plsc

_D = 1024
_E = 8
_K = 2
_H = 2752
_T = 2048
_BLK = 256
_P = _T * _K + _E * _BLK
_NB = _P // _BLK


_CHUNK = 256
_NCHUNK = _T // _CHUNK


def _meta_body(x_ref, wr_ref, dsta_ref, dstb_ref, wa_ref, wb_ref, be_ref):
    logits = jax.lax.dot_general(
        x_ref[...].astype(jnp.bfloat16), wr_ref[...].astype(jnp.bfloat16),
        (((1,), (1,)), ((), ())),
        preferred_element_type=jnp.float32)

    lane = jax.lax.broadcasted_iota(jnp.int32, (_T, _E), 1)
    v1 = jnp.max(logits, axis=1, keepdims=True)
    i1 = jnp.min(jnp.where(logits == v1, lane, _E), axis=1,
                 keepdims=True)
    masked = jnp.where(lane == i1, -jnp.inf, logits)
    v2 = jnp.max(masked, axis=1, keepdims=True)
    i2 = jnp.min(jnp.where(masked == v2, lane, _E), axis=1,
                 keepdims=True)
    wa = 1.0 / (1.0 + jnp.exp(v2 - v1))
    wa_ref[...] = jnp.broadcast_to(wa, (_T, 16))
    wb_ref[...] = jnp.broadcast_to(1.0 - wa, (_T, 16))

    oha = (lane == i1)
    ohb = (lane == i2)
    s = (oha.astype(jnp.float32) + ohb.astype(jnp.float32))
    r = jax.lax.broadcasted_iota(jnp.int32, (_CHUNK, _CHUNK), 0)
    c = jax.lax.broadcasted_iota(jnp.int32, (_CHUNK, _CHUNK), 1)
    tri = (c < r).astype(jnp.bfloat16)
    parts = []
    run = jnp.zeros((1, _E), jnp.float32)
    for ci in range(_NCHUNK):
        sc_ = jax.lax.slice_in_dim(s, ci * _CHUNK, (ci + 1) * _CHUNK, axis=0)
        intra = jax.lax.dot_general(
            tri, sc_.astype(jnp.bfloat16), (((1,), (0,)), ((), ())),
            preferred_element_type=jnp.float32)
        parts.append(intra + run)
        run = run + jnp.sum(sc_, axis=0, keepdims=True)
    cexcl = jnp.concatenate(parts, axis=0)
    gsz = run

    padded = jnp.ceil(gsz / _BLK) * _BLK
    el = jax.lax.broadcasted_iota(jnp.int32, (_E, _E), 1)
    er = jax.lax.broadcasted_iota(jnp.int32, (_E, _E), 0)
    pstart = jnp.sum(jnp.where(el < er, jnp.broadcast_to(padded, (_E, _E)),
                               0.0), axis=1, keepdims=True)
    pstart_row = jnp.broadcast_to(pstart.reshape(1, _E), (_T, _E))

    ranka = jnp.sum(jnp.where(oha, cexcl, 0.0), axis=1, keepdims=True)
    rankb = jnp.sum(jnp.where(ohb, cexcl + oha.astype(jnp.float32), 0.0),
                    axis=1, keepdims=True)
    psa = jnp.sum(jnp.where(oha, pstart_row, 0.0), axis=1, keepdims=True)
    psb = jnp.sum(jnp.where(ohb, pstart_row, 0.0), axis=1, keepdims=True)
    dsta = (psa + ranka).astype(jnp.int32)
    dstb = (psb + rankb).astype(jnp.int32)
    dsta_ref[...] = dsta
    dstb_ref[...] = dstb

    blk = jax.lax.broadcasted_iota(jnp.int32, (_NB, _E), 0) * _BLK
    psrow = jnp.broadcast_to(pstart.reshape(1, _E), (_NB, _E))
    be_ref[...] = (jnp.sum((psrow <= blk.astype(jnp.float32))
                           .astype(jnp.int32), axis=1, keepdims=True) - 1)


def _router_meta(x_flat, Wr):
    return pl.pallas_call(
        _meta_body,
        out_shape=(
            jax.ShapeDtypeStruct((_T, 1), jnp.int32),
            jax.ShapeDtypeStruct((_T, 1), jnp.int32),
            jax.ShapeDtypeStruct((_T, 16), jnp.float32),
            jax.ShapeDtypeStruct((_T, 16), jnp.float32),
            jax.ShapeDtypeStruct((_NB, 1), jnp.int32),
        ),
    )(x_flat, Wr)


_NT = (((1,), (1,)), ((), ()))


def _f1_body(be_ref, xs_ref, w1_ref, w3_ref, g_ref):
    x = xs_ref[...].astype(jnp.bfloat16)
    w1 = w1_ref[0].astype(jnp.bfloat16)
    w3 = w3_ref[0].astype(jnp.bfloat16)
    h1 = jax.lax.dot_general(x, w1, _NT, preferred_element_type=jnp.float32)
    h3 = jax.lax.dot_general(x, w3, _NT, preferred_element_type=jnp.float32)
    g_ref[...] = (h1 * (1.0 / (1.0 + jnp.exp(-h1))) * h3).astype(jnp.bfloat16)


def _f2_body(be_ref, g_ref, w2_ref, out_ref):
    g = g_ref[...]
    w2 = w2_ref[0].astype(jnp.bfloat16)
    out_ref[...] = jax.lax.dot_general(
        g, w2, (((1,), (0,)), ((), ())), preferred_element_type=jnp.float32)


def _ffn(block_expert, xs, W1, W2, W3):
    gs1 = pltpu.PrefetchScalarGridSpec(
        num_scalar_prefetch=1,
        grid=(_NB,),
        in_specs=[
            pl.BlockSpec((_BLK, _D), lambda i, be: (i, 0)),
            pl.BlockSpec((1, _H, _D), lambda i, be: (be[i], 0, 0)),
            pl.BlockSpec((1, _H, _D), lambda i, be: (be[i], 0, 0)),
        ],
        out_specs=pl.BlockSpec((_BLK, _H), lambda i, be: (i, 0)),
    )
    g = pl.pallas_call(
        _f1_body,
        grid_spec=gs1,
        out_shape=jax.ShapeDtypeStruct((_P, _H), jnp.bfloat16),
        compiler_params=pltpu.CompilerParams(
            dimension_semantics=("arbitrary",)),
    )(block_expert, xs, W1, W3)

    gs2 = pltpu.PrefetchScalarGridSpec(
        num_scalar_prefetch=1,
        grid=(_NB,),
        in_specs=[
            pl.BlockSpec((_BLK, _H), lambda i, be: (i, 0)),
            pl.BlockSpec((1, _H, _D), lambda i, be: (be[i], 0, 0)),
        ],
        out_specs=pl.BlockSpec((_BLK, _D), lambda i, be: (i, 0)),
    )
    return pl.pallas_call(
        _f2_body,
        grid_spec=gs2,
        out_shape=jax.ShapeDtypeStruct((_P, _D), jnp.float32),
        compiler_params=pltpu.CompilerParams(
            dimension_semantics=("arbitrary",)),
    )(block_expert, g, jnp.swapaxes(W2, 1, 2))


_NC = 2
_NS = 16
_NW = _NC * _NS
_TPW = _T // _NW
_CH = _TPW // 2

_sc_mesh = plsc.VectorSubcoreMesh(core_axis_name="c", subcore_axis_name="s")


def _sc_scatter_body(x_hbm, dstA_hbm, dstB_hbm, xs_hbm,
                     idxA_v, idxB_v, rows_v, sem):
    wid = lax.axis_index("s") * _NC + lax.axis_index("c")
    base = wid * _TPW
    pltpu.sync_copy(dstA_hbm.at[pl.ds(base, _TPW)], idxA_v)
    pltpu.sync_copy(dstB_hbm.at[pl.ds(base, _TPW)], idxB_v)
    pltpu.sync_copy(x_hbm.at[pl.ds(base, _TPW)], rows_v)
    cpA = pltpu.make_async_copy(rows_v, xs_hbm.at[idxA_v], sem)
    cpB = pltpu.make_async_copy(rows_v, xs_hbm.at[idxB_v], sem)
    cpA.start()
    cpB.start()
    cpA.wait()
    cpB.wait()


@functools.partial(
    pl.kernel,
    out_type=jax.ShapeDtypeStruct((_P, _D), jnp.float32),
    mesh=_sc_mesh,
    scratch_types=[
        pltpu.VMEM((_TPW,), jnp.int32),
        pltpu.VMEM((_TPW,), jnp.int32),
        pltpu.VMEM((_TPW, _D), jnp.float32),
        pltpu.SemaphoreType.DMA,
    ],
)
def _sc_scatter(x_hbm, dstA_hbm, dstB_hbm, xs_hbm, idxA_v, idxB_v, rows_v, sem):
    _sc_scatter_body(x_hbm, dstA_hbm, dstB_hbm, xs_hbm,
                     idxA_v, idxB_v, rows_v, sem)


_CC = 16
_NCC = _TPW // _CC


def _sc_combine_body(ys_hbm, dstA_hbm, dstB_hbm, wA_hbm, wB_hbm, out_hbm,
                     idxA_v, idxB_v, wA_v, wB_v, bufA_v, bufB_v, out_v, sem):
    wid = lax.axis_index("s") * _NC + lax.axis_index("c")
    base = wid * _TPW
    pltpu.sync_copy(dstA_hbm.at[pl.ds(base, _TPW)], idxA_v)
    pltpu.sync_copy(dstB_hbm.at[pl.ds(base, _TPW)], idxB_v)
    pltpu.sync_copy(wA_hbm.at[pl.ds(base, _TPW)], wA_v)
    pltpu.sync_copy(wB_hbm.at[pl.ds(base, _TPW)], wB_v)

    def gathers(ch):
        par = ch % 2
        cpA = pltpu.make_async_copy(
            ys_hbm.at[idxA_v.at[pl.ds(ch * _CC, _CC)]], bufA_v.at[par], sem)
        cpB = pltpu.make_async_copy(
            ys_hbm.at[idxB_v.at[pl.ds(ch * _CC, _CC)]], bufB_v.at[par], sem)
        cpA.start()
        cpB.start()
        return cpA, cpB

    pend = gathers(0)
    for ch in range(_NCC):
        par = ch % 2
        pend[0].wait()
        pend[1].wait()
        if ch + 1 < _NCC:
            pend = gathers(ch + 1)

        def tok(i, _):
            wa = wA_v[ch * _CC + i, :]
            wb = wB_v[ch * _CC + i, :]
            for k in range(_D // 16):
                a = bufA_v[par, i, pl.ds(k * 16, 16)]
                b = bufB_v[par, i, pl.ds(k * 16, 16)]
                out_v[i, pl.ds(k * 16, 16)] = wa * a + wb * b
            return 0

        lax.fori_loop(0, _CC, tok, 0)
        pltpu.sync_copy(out_v, out_hbm.at[pl.ds(base + ch * _CC, _CC)])


@functools.partial(
    pl.kernel,
    out_type=jax.ShapeDtypeStruct((_T, _D), jnp.float32),
    mesh=_sc_mesh,
    scratch_types=[
        pltpu.VMEM((_TPW,), jnp.int32),
        pltpu.VMEM((_TPW,), jnp.int32),
        pltpu.VMEM((_TPW, 16), jnp.float32),
        pltpu.VMEM((_TPW, 16), jnp.float32),
        pltpu.VMEM((2, _CC, _D), jnp.float32),
        pltpu.VMEM((2, _CC, _D), jnp.float32),
        pltpu.VMEM((_CC, _D), jnp.float32),
        pltpu.SemaphoreType.DMA,
    ],
)
def _sc_combine(ys_hbm, dstA_hbm, dstB_hbm, wA_hbm, wB_hbm, out_hbm,
                idxA_v, idxB_v, wA_v, wB_v, bufA_v, bufB_v, out_v, sem):
    _sc_combine_body(ys_hbm, dstA_hbm, dstB_hbm, wA_hbm, wB_hbm, out_hbm,
                     idxA_v, idxB_v, wA_v, wB_v, bufA_v, bufB_v, out_v, sem)


def kernel(x, Wr, W1, W2, W3):
    x_flat = x.reshape(_T, _D)
    dstA2, dstB2, wA, wB, be2 = _router_meta(x_flat, Wr)
    dstA = dstA2.reshape(_T)
    dstB = dstB2.reshape(_T)
    block_expert = be2.reshape(_NB)

    xs = _sc_scatter(x_flat, dstA, dstB)
    ys = _ffn(block_expert, xs, W1, W2, W3)
    out = _sc_combine(ys, dstA, dstB, wA, wB)
    return out.reshape(1, _T, _D)
